# baseline (speedup 1.0000x reference)
import functools

import jax
import jax.numpy as jnp
from jax import lax
from jax.experimental import pallas as pl
from jax.experimental.pallas import tpu as pltpu
from jax.experimental.pallas import tpu_sc as plsc

P_LEFT = 0.12
P_RIGHT = 0.12
P_MIDDLE = 1.0 - P_LEFT - P_RIGHT

_NUM_CORES = 2
_NUM_SUBCORES = 16
_LANES = 16
_NW = _NUM_CORES * _NUM_SUBCORES


def _build_tinds(T):
    skey = jax.random.key(42)
    logits = jnp.log(jnp.array([P_LEFT, P_MIDDLE, P_RIGHT], dtype=jnp.float32))
    jitters = jax.random.categorical(skey, logits, shape=(T,))
    tinds = jnp.arange(T, dtype=jnp.int32) + jitters.astype(jnp.int32)
    return jnp.clip(tinds, 0, T - 1)


@functools.partial(jax.jit, static_argnames=("n_rows", "T", "unroll"))
def _sc_jitter_gather(x2, tind, n_rows, T, unroll):
    rows_per_w = n_rows // _NW
    rb = 8
    hb = rb // 2
    nblk = rows_per_w // rb
    ngrp = T // _LANES
    mesh = plsc.VectorSubcoreMesh(core_axis_name="c", subcore_axis_name="s")

    @functools.partial(
        pl.kernel,
        out_type=jax.ShapeDtypeStruct((n_rows, T), jnp.float32),
        mesh=mesh,
        scratch_types=[
            pltpu.VMEM((T,), jnp.int32),
            pltpu.VMEM((rb, T), jnp.float32),
            pltpu.VMEM((rb, T), jnp.float32),
            pltpu.VMEM((rb, T // 2), jnp.float32),
            pltpu.VMEM((rb, T // 2), jnp.float32),
            pltpu.SemaphoreType.DMA,
            pltpu.SemaphoreType.DMA,
            pltpu.SemaphoreType.DMA,
            pltpu.SemaphoreType.DMA,
        ],
        compiler_params=pltpu.CompilerParams(
            needs_layout_passes=False, use_tc_tiling_on_sc=True,
            disable_bounds_checks=True, disable_semaphore_checks=True,
            skip_device_barrier=True
        ),
    )
    def k(x_hbm, tind_hbm, out_hbm, idx_v, in0, in1, oa, ob, is0, is1, osa, osb):
        wid = lax.axis_index("s") * _NUM_CORES + lax.axis_index("c")
        w0 = wid * rows_per_w
        pltpu.sync_copy(tind_hbm, idx_v)

        ins = (in0, in1)
        isems = (is0, is1)

        def in_copy(b, p):
            return pltpu.make_async_copy(
                x_hbm.at[pl.ds(w0 + b * rb, rb), :], ins[p], isems[p]
            )

        def oa_copy(b):
            return pltpu.make_async_copy(
                oa, out_hbm.at[pl.ds(w0 + b * rb, rb), pl.ds(0, T // 2)], osa
            )

        def ob_copy(b):
            return pltpu.make_async_copy(
                ob, out_hbm.at[pl.ds(w0 + b * rb, rb), pl.ds(T // 2, T // 2)], osb
            )

        def compute_half(p, half, dst):
            in_v = ins[p]
            cbase = half * (ngrp // 2)

            @plsc.parallel_loop(0, ngrp // 2, 1, unroll=unroll)
            def grp(g):
                off = pl.multiple_of(g * _LANES, _LANES)
                idx = idx_v[pl.ds(cbase * _LANES + off, _LANES)]
                for r in range(rb):
                    rvec = jnp.full((_LANES,), r, jnp.int32)
                    dst[r, pl.ds(off, _LANES)] = plsc.load_gather(
                        in_v, [rvec, idx]
                    )

        in_copy(0, 0).start()
        in_copy(1, 1).start()

        def pair_body(j, carry):
            for p in range(2):
                b = j * 2 + p
                in_copy(b, p).wait()

                @pl.when(b > 0)
                def _():
                    oa_copy(b - 1).wait()

                compute_half(p, 0, oa)
                oa_copy(b).start()

                @pl.when(b > 0)
                def _():
                    ob_copy(b - 1).wait()

                compute_half(p, 1, ob)
                ob_copy(b).start()

                @pl.when(b + 2 < nblk)
                def _():
                    in_copy(b + 2, p).start()

            return carry

        lax.fori_loop(0, nblk // 2, pair_body, 0)
        oa_copy(nblk - 1).wait()
        ob_copy(nblk - 1).wait()

    return k(x2, tind)


def kernel(x):
    T = x.shape[-1]
    n_rows = x.size // T
    tind = _build_tinds(T)
    out = _sc_jitter_gather(x.reshape(n_rows, T), tind, n_rows=n_rows, T=T, unroll=4)
    return out.reshape(x.shape)


# speedup vs baseline: 1.0023x; 1.0023x over previous
import functools

import jax
import jax.numpy as jnp
from jax import lax
from jax.experimental import pallas as pl
from jax.experimental.pallas import tpu as pltpu
from jax.experimental.pallas import tpu_sc as plsc

P_LEFT = 0.12
P_RIGHT = 0.12
P_MIDDLE = 1.0 - P_LEFT - P_RIGHT

_NUM_CORES = 2
_NUM_SUBCORES = 16
_LANES = 16
_NW = _NUM_CORES * _NUM_SUBCORES


def _build_tinds(T):
    skey = jax.random.key(42)
    logits = jnp.log(jnp.array([P_LEFT, P_MIDDLE, P_RIGHT], dtype=jnp.float32))
    jitters = jax.random.categorical(skey, logits, shape=(T,))
    tinds = jnp.arange(T, dtype=jnp.int32) + jitters.astype(jnp.int32)
    return jnp.clip(tinds, 0, T - 1)


@functools.partial(jax.jit, static_argnames=("n_rows", "T", "unroll"))
def _sc_jitter_gather(x2, tind, n_rows, T, unroll):
    rows_per_w = n_rows // _NW
    rb = 8
    hb = rb // 2
    nblk = rows_per_w // rb
    ngrp = T // _LANES
    mesh = plsc.VectorSubcoreMesh(core_axis_name="c", subcore_axis_name="s")

    @functools.partial(
        pl.kernel,
        out_type=jax.ShapeDtypeStruct((n_rows, T), jnp.float32),
        mesh=mesh,
        scratch_types=[
            pltpu.VMEM((T,), jnp.int32),
            pltpu.VMEM((rb, T), jnp.float32),
            pltpu.VMEM((rb, T), jnp.float32),
            pltpu.VMEM((rb, T // 2), jnp.float32),
            pltpu.VMEM((rb, T // 2), jnp.float32),
            pltpu.SemaphoreType.DMA,
            pltpu.SemaphoreType.DMA,
            pltpu.SemaphoreType.DMA,
            pltpu.SemaphoreType.DMA,
        ],
        compiler_params=pltpu.CompilerParams(
            needs_layout_passes=False, use_tc_tiling_on_sc=True
        ),
    )
    def k(x_hbm, tind_hbm, out_hbm, idx_v, in0, in1, oa, ob, is0, is1, osa, osb):
        wid = lax.axis_index("s") * _NUM_CORES + lax.axis_index("c")
        w0 = wid * rows_per_w
        pltpu.sync_copy(tind_hbm, idx_v)

        ins = (in0, in1)
        isems = (is0, is1)

        def in_copy(b, p):
            return pltpu.make_async_copy(
                x_hbm.at[pl.ds(w0 + b * rb, rb), :], ins[p], isems[p]
            )

        def oa_copy(b):
            return pltpu.make_async_copy(
                oa, out_hbm.at[pl.ds(w0 + b * rb, rb), pl.ds(0, T // 2)], osa
            )

        def ob_copy(b):
            return pltpu.make_async_copy(
                ob, out_hbm.at[pl.ds(w0 + b * rb, rb), pl.ds(T // 2, T // 2)], osb
            )

        def compute_half(p, half, dst):
            in_v = ins[p]
            cbase = half * (ngrp // 2)

            @plsc.parallel_loop(0, ngrp // 2, 1, unroll=unroll)
            def grp(g):
                off = pl.multiple_of(g * _LANES, _LANES)
                idx = idx_v[pl.ds(cbase * _LANES + off, _LANES)]
                for r in range(rb):
                    rvec = jnp.full((_LANES,), r, jnp.int32)
                    dst[r, pl.ds(off, _LANES)] = plsc.load_gather(
                        in_v, [rvec, idx]
                    )

        in_copy(0, 0).start()
        in_copy(1, 1).start()

        def pair_body(j, carry):
            for p in range(2):
                b = j * 2 + p
                in_copy(b, p).wait()

                @pl.when(b > 0)
                def _():
                    oa_copy(b - 1).wait()

                compute_half(p, 0, oa)
                oa_copy(b).start()

                @pl.when(b > 0)
                def _():
                    ob_copy(b - 1).wait()

                compute_half(p, 1, ob)
                ob_copy(b).start()

                @pl.when(b + 2 < nblk)
                def _():
                    in_copy(b + 2, p).start()

            return carry

        lax.fori_loop(0, nblk // 2, pair_body, 0)
        oa_copy(nblk - 1).wait()
        ob_copy(nblk - 1).wait()

    return k(x2, tind)


def kernel(x):
    T = x.shape[-1]
    n_rows = x.size // T
    tind = _build_tinds(T)
    out = _sc_jitter_gather(x.reshape(n_rows, T), tind, n_rows=n_rows, T=T, unroll=4)
    return out.reshape(x.shape)


# speedup vs baseline: 1.0055x; 1.0032x over previous
import functools

import jax
import jax.numpy as jnp
from jax import lax
from jax.experimental import pallas as pl
from jax.experimental.pallas import tpu as pltpu
from jax.experimental.pallas import tpu_sc as plsc

P_LEFT = 0.12
P_RIGHT = 0.12
P_MIDDLE = 1.0 - P_LEFT - P_RIGHT

_NUM_CORES = 2
_NUM_SUBCORES = 16
_LANES = 16
_NW = _NUM_CORES * _NUM_SUBCORES


def _build_tinds(T):
    skey = jax.random.key(42)
    logits = jnp.log(jnp.array([P_LEFT, P_MIDDLE, P_RIGHT], dtype=jnp.float32))
    jitters = jax.random.categorical(skey, logits, shape=(T,))
    tinds = jnp.arange(T, dtype=jnp.int32) + jitters.astype(jnp.int32)
    return jnp.clip(tinds, 0, T - 1)


@functools.partial(jax.jit, static_argnames=("n_rows", "T", "unroll"))
def _sc_jitter_gather(x2, tind, n_rows, T, unroll):
    rows_per_w = n_rows // _NW
    rb = 8
    nblk = rows_per_w // rb
    ngrp = T // _LANES
    mesh = plsc.VectorSubcoreMesh(core_axis_name="c", subcore_axis_name="s")

    @functools.partial(
        pl.kernel,
        out_type=jax.ShapeDtypeStruct((n_rows, T), jnp.float32),
        mesh=mesh,
        scratch_types=[
            pltpu.VMEM((T,), jnp.int32),
            pltpu.VMEM((rb, T), jnp.float32),
            pltpu.VMEM((rb, T), jnp.float32),
            pltpu.VMEM((rb, T // 2), jnp.float32),
            pltpu.VMEM((rb, T // 2), jnp.float32),
            pltpu.SemaphoreType.DMA,
            pltpu.SemaphoreType.DMA,
            pltpu.SemaphoreType.DMA,
            pltpu.SemaphoreType.DMA,
        ],
        compiler_params=pltpu.CompilerParams(
            needs_layout_passes=False, use_tc_tiling_on_sc=True
        ),
    )
    def k(x_hbm, tind_hbm, out_hbm, idx_v, in0, in1, oa, ob, is0, is1, osa, osb):
        wid = lax.axis_index("s") * _NUM_CORES + lax.axis_index("c")
        w0 = wid * rows_per_w
        pltpu.sync_copy(tind_hbm, idx_v)

        ins = (in0, in1)
        isems = (is0, is1)

        def in_copy(b, p):
            return pltpu.make_async_copy(
                x_hbm.at[pl.ds(w0 + b * rb, rb), :], ins[p], isems[p]
            )

        def oa_copy(b):
            return pltpu.make_async_copy(
                oa, out_hbm.at[pl.ds(w0 + b * rb, rb), pl.ds(0, T // 2)], osa
            )

        def ob_copy(b):
            return pltpu.make_async_copy(
                ob, out_hbm.at[pl.ds(w0 + b * rb, rb), pl.ds(T // 2, T // 2)], osb
            )

        def compute_half(p, half, dst):
            in_v = ins[p]
            cbase = half * (ngrp // 2)

            @plsc.parallel_loop(0, ngrp // 2, 1, unroll=unroll)
            def grp(g):
                off = pl.multiple_of(g * _LANES, _LANES)
                idx = idx_v[pl.ds(cbase * _LANES + off, _LANES)]
                for r in range(rb):
                    rvec = jnp.full((_LANES,), r, jnp.int32)
                    dst[r, pl.ds(off, _LANES)] = plsc.load_gather(
                        in_v, [rvec, idx]
                    )

        in_copy(0, 0).start()
        in_copy(1, 1).start()

        def pair_body(j, carry):
            for p in range(2):
                b = j * 2 + p
                in_copy(b, p).wait()

                @pl.when(b > 0)
                def _():
                    oa_copy(b - 1).wait()

                compute_half(p, 0, oa)
                oa_copy(b).start()

                @pl.when(b > 0)
                def _():
                    ob_copy(b - 1).wait()

                compute_half(p, 1, ob)
                ob_copy(b).start()

                @pl.when(b + 2 < nblk)
                def _():
                    in_copy(b + 2, p).start()

            return carry

        lax.fori_loop(0, nblk // 2, pair_body, 0)
        oa_copy(nblk - 1).wait()
        ob_copy(nblk - 1).wait()

    return k(x2, tind)


def kernel(x):
    T = x.shape[-1]
    n_rows = x.size // T
    tind = _build_tinds(T)
    out = _sc_jitter_gather(x.reshape(n_rows, T), tind, n_rows=n_rows, T=T, unroll=4)
    return out.reshape(x.shape)
